# baseline (speedup 1.0000x reference)
import jax
import jax.numpy as jnp
import numpy as np
from jax.experimental import pallas as pl

N = 10000
E = 320000
D_IN = 128
D_ATTR = 16
D_EDGE = 4
D_OUT = 64
MUL = 64
HID = 288
NB = 10
MAXR = 5.0
NUM_NEIGH = 32.0
DIMS = [D_IN, HID, HID, HID, D_OUT]
EPS = 1e-12


def _sph_harm_lmax2(vec):
    n = jnp.sqrt(jnp.sum(vec * vec, axis=1, keepdims=True) + EPS)
    v = vec / n
    x_, y_, z_ = v[:, 0], v[:, 1], v[:, 2]
    sh0 = jnp.ones_like(x_)[:, None]
    sh1 = jnp.sqrt(3.0) * v
    c15 = jnp.sqrt(15.0)
    c5 = jnp.sqrt(5.0)
    sh2 = jnp.stack([
        c15 * x_ * z_,
        c15 * x_ * y_,
        c5 * (y_ * y_ - 0.5 * (x_ * x_ + z_ * z_)),
        c15 * y_ * z_,
        (c15 / 2.0) * (z_ * z_ - x_ * x_),
    ], axis=1)
    return jnp.concatenate([sh0, sh1, sh2], axis=1)


def _soft_one_hot_cosine(x, start, end, number):
    values = jnp.linspace(start, end, number + 2)[1:-1]
    step = values[1] - values[0]
    diff = (x[:, None] - values[None, :]) / step
    y = jnp.cos(0.5 * jnp.pi * diff)
    mask = ((diff > -1.0) & (diff < 1.0)).astype(x.dtype)
    return y * mask


def _copy_kernel(x_ref, o_ref):
    o_ref[...] = x_ref[...]


def kernel(x, pos, node_attr, edge_attr, params, edge_index, batch):
    edge_src = edge_index[0]
    edge_dst = edge_index[1]
    edge_vec = pos[edge_src] - pos[edge_dst]
    edge_sh = _sph_harm_lmax2(edge_vec)
    ea_full = jnp.concatenate([edge_attr, edge_sh], axis=1)
    edge_len = jnp.sqrt(jnp.sum(edge_vec * edge_vec, axis=1) + EPS)
    emb = _soft_one_hot_cosine(edge_len, 0.0, MAXR, NB) * (NB ** 0.5)
    h = x
    for i in range(4):
        r = jax.nn.silu(emb @ params['W1_%d' % i] + params['b1_%d' % i])
        r = jax.nn.silu(r @ params['W2_%d' % i] + params['b2_%d' % i])
        r = r @ params['W_rad_%d' % i]
        hm = h @ params['W_msg_%d' % i]
        msg = hm[edge_src] * (ea_full @ params['W_sh_%d' % i]) * r
        agg = jax.ops.segment_sum(msg, edge_dst, num_segments=N) / jnp.sqrt(NUM_NEIGH)
        h = agg + h @ params['W_self_%d' % i] + node_attr @ params['W_attr_%d' % i]
        if i < 3:
            h = jax.nn.silu(h)
    h = pl.pallas_call(
        _copy_kernel,
        out_shape=jax.ShapeDtypeStruct(h.shape, h.dtype),
    )(h)
    return (h, node_attr, edge_src, edge_dst, ea_full, emb, batch)


# speedup vs baseline: 2.0309x; 2.0309x over previous
import functools

import jax
import jax.numpy as jnp
import numpy as np
from jax import lax
from jax.experimental import pallas as pl
from jax.experimental.pallas import tpu as pltpu
from jax.experimental.pallas import tpu_sc as plsc

N = 10000
E = 320000
D_IN = 128
D_ATTR = 16
D_EDGE = 4
D_OUT = 64
MUL = 64
HID = 288
NB = 10
MAXR = 5.0
NUM_NEIGH = 32.0
DIMS = [D_IN, HID, HID, HID, D_OUT]
EPS = 1e-12

NC = 2
NS = 16
CH = 128
LW = 128

_PACK = {HID: (3, (128, 128, 32)), D_OUT: (1, (64,))}

_MESH = dict(core_axis_name="c", subcore_axis_name="s", num_cores=NC,
             num_subcores=NS)


_NW = NC * NS
_PG_CHUNKS = E // CH


def _edge_vec_body(pos128, srcdst, vec_out, si_v, di_v, rs_v, rd_v, pk_v,
                   sem):
    cid = lax.axis_index("c")
    sid = lax.axis_index("s")
    wid = sid * NC + cid
    nch = jnp.where(wid < _PG_CHUNKS % _NW, _PG_CHUNKS // _NW + 1,
                    _PG_CHUNKS // _NW)

    def kbody(k, carry):
        off = (k * _NW + wid) * CH
        pltpu.sync_copy(srcdst.at[pl.ds(off, CH)], si_v)
        pltpu.sync_copy(srcdst.at[pl.ds(E + off, CH)], di_v)
        pltpu.async_copy(pos128.at[si_v], rs_v, sem).wait()
        pltpu.async_copy(pos128.at[di_v], rd_v, sem).wait()

        def pack_row(i, carry2):
            r8 = i // 8
            c16 = (i % 8) * 16
            pk_v[r8, pl.ds(c16, 16)] = (rs_v[i, pl.ds(0, 16)]
                                        - rd_v[i, pl.ds(0, 16)])
            return carry2

        lax.fori_loop(0, CH, pack_row, 0)
        ro = pl.multiple_of((k * _NW + wid) * (CH // 8), 8)
        pltpu.sync_copy(pk_v, vec_out.at[pl.ds(ro, CH // 8)])
        return carry

    lax.fori_loop(0, nch, kbody, 0)


@functools.cache
def _edge_vec():
    return pl.kernel(
        _edge_vec_body,
        out_type=jax.ShapeDtypeStruct((E // 8, LW), jnp.float32),
        mesh=plsc.VectorSubcoreMesh(**_MESH),
        scratch_types=[
            pltpu.VMEM((CH,), jnp.int32),
            pltpu.VMEM((CH,), jnp.int32),
            pltpu.VMEM((CH, LW), jnp.float32),
            pltpu.VMEM((CH, LW), jnp.float32),
            pltpu.VMEM((CH // 8, LW), jnp.float32),
            pltpu.SemaphoreType.DMA,
        ],
    )


E2 = E // NC
L_CHUNKS = E2 // CH

RS = 624
_RS_EXTRA_OFF = RS * NS
_RS_EXTRA = N - _RS_EXTRA_OFF


def _rows_copy(src_at, dst_at, sid):
    pltpu.sync_copy(src_at(sid * RS, RS), dst_at(sid * RS, RS))

    @pl.when(sid == NS - 1)
    def _():
        pltpu.sync_copy(src_at(_RS_EXTRA_OFF, _RS_EXTRA),
                        dst_at(_RS_EXTRA_OFF, _RS_EXTRA))


def _make_layer_sc(nblk, used):
    nch_lo = L_CHUNKS // NS
    nch_rem = L_CHUNKS % NS

    def body(hm3, w3, src3, dst, zeros, out, gi_v, di_v, rows_v, w_v, agg_sp,
             sem):
        cid = lax.axis_index("c")
        sid = lax.axis_index("s")
        nch = jnp.where(sid < nch_rem, nch_lo + 1, nch_lo)

        for cb in range(nblk):
            nlane = (used[cb] + 15) // 16

            _rows_copy(lambda o, n: zeros.at[pl.ds(o, n)],
                       lambda o, n: agg_sp.at[pl.ds(o, n)], sid)
            plsc.subcore_barrier()

            def kbody(k, carry, _cb=cb, _nlane=nlane):
                off = cid * E2 + (k * NS + sid) * CH
                pltpu.sync_copy(src3.at[pl.ds(_cb * E + off, CH)], gi_v)
                pltpu.sync_copy(dst.at[pl.ds(off, CH)], di_v)
                pltpu.async_copy(hm3.at[gi_v], rows_v, sem).wait()
                pltpu.sync_copy(w3.at[pl.ds(_cb * E + off, CH)], w_v)

                def mul_row(i, carry2):
                    for j in range(_nlane):
                        sl = pl.ds(j * 16, 16)
                        rows_v[i, sl] = rows_v[i, sl] * w_v[i, sl]
                    return carry2

                lax.fori_loop(0, CH, mul_row, 0)
                pltpu.sync_copy(rows_v, agg_sp.at[di_v], add=True)
                return carry

            lax.fori_loop(0, nch, kbody, 0)
            plsc.subcore_barrier()
            obase = (cid * nblk + cb) * N
            _rows_copy(lambda o, n: agg_sp.at[pl.ds(o, n)],
                       lambda o, n: out.at[pl.ds(obase + o, n)], sid)
            plsc.subcore_barrier()

    return pl.kernel(
        body,
        out_type=jax.ShapeDtypeStruct((NC * nblk * N, LW), jnp.float32),
        mesh=plsc.VectorSubcoreMesh(**_MESH),
        scratch_types=[
            pltpu.VMEM((CH,), jnp.int32),
            pltpu.VMEM((CH,), jnp.int32),
            pltpu.VMEM((CH, LW), jnp.float32),
            pltpu.VMEM((CH, LW), jnp.float32),
            pltpu.VMEM_SHARED((N, LW), jnp.float32),
            pltpu.SemaphoreType.DMA,
        ],
    )


@functools.cache
def _layer_sc(do):
    nblk, used = _PACK[do]
    return _make_layer_sc(nblk, used)


BE = 3200

_SQ3 = float(np.sqrt(3.0))
_SQ15 = float(np.sqrt(15.0))
_SQ5 = float(np.sqrt(5.0))
_EMB_STEP = float(MAXR / (NB + 1))
_EMB_SCALE = float(NB ** 0.5)


def _geom_body(vecp_ref, eattr_ref, ea_ref, emb_ref):
    vec = vecp_ref[:, :3]
    norm = jnp.sqrt(jnp.sum(vec * vec, axis=1, keepdims=True) + EPS)
    v = vec / norm
    x_, y_, z_ = v[:, 0], v[:, 1], v[:, 2]
    sh2 = jnp.stack([
        _SQ15 * x_ * z_,
        _SQ15 * x_ * y_,
        _SQ5 * (y_ * y_ - 0.5 * (x_ * x_ + z_ * z_)),
        _SQ15 * y_ * z_,
        (_SQ15 / 2.0) * (z_ * z_ - x_ * x_),
    ], axis=1)
    ea_ref[...] = jnp.concatenate(
        [eattr_ref[...], jnp.ones_like(norm), _SQ3 * v, sh2], axis=1)
    values = (lax.broadcasted_iota(jnp.int32, (1, NB), 1).astype(jnp.float32)
              + 1.0) * _EMB_STEP
    diff = (norm - values) / _EMB_STEP
    y = jnp.cos(0.5 * np.pi * diff)
    mask = ((diff > -1.0) & (diff < 1.0)).astype(jnp.float32)
    emb_ref[...] = y * mask * _EMB_SCALE


def _geom(vecp, edge_attr):
    return pl.pallas_call(
        _geom_body,
        grid=(E // BE,),
        in_specs=[
            pl.BlockSpec((BE, 16), lambda i: (i, 0)),
            pl.BlockSpec((BE, D_EDGE), lambda i: (i, 0)),
        ],
        out_specs=[
            pl.BlockSpec((BE, D_EDGE + 9), lambda i: (i, 0)),
            pl.BlockSpec((BE, NB), lambda i: (i, 0)),
        ],
        out_shape=[
            jax.ShapeDtypeStruct((E, D_EDGE + 9), jnp.float32),
            jax.ShapeDtypeStruct((E, NB), jnp.float32),
        ],
    )(vecp, edge_attr)


def _ew_body(emb_ref, ea_ref, w1_ref, b1_ref, w2_ref, b2_ref, wr_ref, ws_ref,
             w_out):
    nblk, _, _ = w_out.shape
    do = wr_ref.shape[1]
    r = jax.nn.silu(emb_ref[...] @ w1_ref[...] + b1_ref[...])
    r = jax.nn.silu(r @ w2_ref[...] + b2_ref[...])
    r = r @ wr_ref[...]
    w = (ea_ref[...] @ ws_ref[...]) * r
    if nblk * LW != do:
        w = jnp.pad(w, ((0, 0), (0, nblk * LW - do)))
    for q in range(nblk):
        w_out[q] = w[:, q * LW:(q + 1) * LW]


def _edge_weights(emb, ea_full, p, i):
    do = DIMS[i + 1]
    nblk, _ = _PACK[do]
    full = lambda i_: (0, 0)
    return pl.pallas_call(
        _ew_body,
        grid=(E // BE,),
        in_specs=[
            pl.BlockSpec((BE, NB), lambda i_: (i_, 0)),
            pl.BlockSpec((BE, D_EDGE + 9), lambda i_: (i_, 0)),
            pl.BlockSpec((NB, MUL), full),
            pl.BlockSpec((1, MUL), full),
            pl.BlockSpec((MUL, 2 * MUL), full),
            pl.BlockSpec((1, 2 * MUL), full),
            pl.BlockSpec((2 * MUL, do), full),
            pl.BlockSpec((D_EDGE + 9, do), full),
        ],
        out_specs=pl.BlockSpec((nblk, BE, LW), lambda i_: (0, i_, 0)),
        out_shape=jax.ShapeDtypeStruct((nblk, E, LW), jnp.float32),
    )(emb, ea_full, p['W1_%d' % i], p['b1_%d' % i].reshape(1, -1),
      p['W2_%d' % i], p['b2_%d' % i].reshape(1, -1), p['W_rad_%d' % i],
      p['W_sh_%d' % i])


BN = 2000
_INV_SQRT_NEIGH = float(1.0 / np.sqrt(NUM_NEIGH))


def _sum_agg(agg_ref, do):
    nblk, _ = _PACK[do]
    cols = []
    for q in range(nblk):
        cols.append(agg_ref[0, q] + agg_ref[1, q])
    agg = jnp.concatenate(cols, axis=1) if nblk > 1 else cols[0]
    return agg[:, :do]


def _pack_hm(hm, nblk, hm_out):
    do = hm.shape[1]
    if nblk * LW != do:
        hm = jnp.pad(hm, ((0, 0), (0, nblk * LW - do)))
    for q in range(nblk):
        hm_out[q] = hm[:, q * LW:(q + 1) * LW]


def _node0_body(x_ref, wm_ref, hm_out):
    _pack_hm(x_ref[...] @ wm_ref[...], hm_out.shape[0], hm_out)


def _node0(x, wm):
    do = wm.shape[1]
    nblk, _ = _PACK[do]
    return pl.pallas_call(
        _node0_body,
        grid=(N // BN,),
        in_specs=[
            pl.BlockSpec((BN, D_IN), lambda i: (i, 0)),
            pl.BlockSpec((D_IN, do), lambda i: (0, 0)),
        ],
        out_specs=pl.BlockSpec((nblk, BN, LW), lambda i: (0, i, 0)),
        out_shape=jax.ShapeDtypeStruct((nblk, N, LW), jnp.float32),
    )(x, wm)


def _node_post_body(agg_ref, h_ref, na_ref, wself_ref, wattr_ref, wm_ref,
                    h_out, hm_out):
    do = wself_ref.shape[1]
    agg = _sum_agg(agg_ref, do)
    hn = (agg * _INV_SQRT_NEIGH + h_ref[...] @ wself_ref[...]
          + na_ref[...] @ wattr_ref[...])
    hn = jax.nn.silu(hn)
    h_out[...] = hn
    _pack_hm(hn @ wm_ref[...], hm_out.shape[0], hm_out)


def _node_post(agg4, h, na, wself, wattr, wm_next):
    di, do = wself.shape
    nblk, _ = _PACK[do]
    don = wm_next.shape[1]
    nblkn, _ = _PACK[don]
    full = lambda i: (0, 0)
    return pl.pallas_call(
        _node_post_body,
        grid=(N // BN,),
        in_specs=[
            pl.BlockSpec((NC, nblk, BN, LW), lambda i: (0, 0, i, 0)),
            pl.BlockSpec((BN, di), lambda i: (i, 0)),
            pl.BlockSpec((BN, D_ATTR), lambda i: (i, 0)),
            pl.BlockSpec((di, do), full),
            pl.BlockSpec((D_ATTR, do), full),
            pl.BlockSpec((do, don), full),
        ],
        out_specs=[
            pl.BlockSpec((BN, do), lambda i: (i, 0)),
            pl.BlockSpec((nblkn, BN, LW), lambda i: (0, i, 0)),
        ],
        out_shape=[
            jax.ShapeDtypeStruct((N, do), jnp.float32),
            jax.ShapeDtypeStruct((nblkn, N, LW), jnp.float32),
        ],
    )(agg4, h, na, wself, wattr, wm_next)


def _node_final_body(agg_ref, h_ref, na_ref, wself_ref, wattr_ref, h_out):
    do = wself_ref.shape[1]
    agg = _sum_agg(agg_ref, do)
    h_out[...] = (agg * _INV_SQRT_NEIGH + h_ref[...] @ wself_ref[...]
                  + na_ref[...] @ wattr_ref[...])


def _node_final(agg4, h, na, wself, wattr):
    di, do = wself.shape
    nblk, _ = _PACK[do]
    full = lambda i: (0, 0)
    return pl.pallas_call(
        _node_final_body,
        grid=(N // BN,),
        in_specs=[
            pl.BlockSpec((NC, nblk, BN, LW), lambda i: (0, 0, i, 0)),
            pl.BlockSpec((BN, di), lambda i: (i, 0)),
            pl.BlockSpec((BN, D_ATTR), lambda i: (i, 0)),
            pl.BlockSpec((di, do), full),
            pl.BlockSpec((D_ATTR, do), full),
        ],
        out_specs=pl.BlockSpec((BN, do), lambda i: (i, 0)),
        out_shape=jax.ShapeDtypeStruct((N, do), jnp.float32),
    )(agg4, h, na, wself, wattr)


def kernel(x, pos, node_attr, edge_attr, params, edge_index, batch):
    src = edge_index[0]
    dst = edge_index[1]
    pos128 = jnp.pad(pos, ((0, 0), (0, LW - 3)))
    src3 = jnp.concatenate([src, src + N, src + 2 * N])
    zeros = jnp.zeros((N, LW), jnp.float32)

    vecp = _edge_vec()(pos128, edge_index.reshape(2 * E))
    ea_full, emb = _geom(vecp.reshape(E, 16), edge_attr)

    h = x
    hm = _node0(x, params['W_msg_0'])
    for i in range(4):
        do = DIMS[i + 1]
        nblk, _ = _PACK[do]
        w3 = _edge_weights(emb, ea_full, params, i)
        agg = _layer_sc(do)(hm.reshape(nblk * N, LW),
                            w3.reshape(nblk * E, LW), src3, dst, zeros)
        agg4 = agg.reshape(NC, nblk, N, LW)
        if i < 3:
            h, hm = _node_post(agg4, h, node_attr, params['W_self_%d' % i],
                               params['W_attr_%d' % i],
                               params['W_msg_%d' % (i + 1)])
        else:
            h = _node_final(agg4, h, node_attr, params['W_self_%d' % i],
                            params['W_attr_%d' % i])

    return (h, node_attr, src, dst, ea_full, emb, batch)


# speedup vs baseline: 2.5306x; 1.2460x over previous
import functools

import jax
import jax.numpy as jnp
import numpy as np
from jax import lax
from jax.experimental import pallas as pl
from jax.experimental.pallas import tpu as pltpu
from jax.experimental.pallas import tpu_sc as plsc

N = 10000
E = 320000
D_IN = 128
D_ATTR = 16
D_EDGE = 4
D_OUT = 64
MUL = 64
HID = 288
NB = 10
MAXR = 5.0
NUM_NEIGH = 32.0
DIMS = [D_IN, HID, HID, HID, D_OUT]
EPS = 1e-12

NC = 2
NS = 16
CH = 128
LW = 128

_PACK = {HID: (3, (128, 128, 32)), D_OUT: (1, (64,))}

_MESH = dict(core_axis_name="c", subcore_axis_name="s", num_cores=NC,
             num_subcores=NS)


_NW = NC * NS
_PG_CHUNKS = E // CH


def _edge_vec_body(pos128, srcdst, vec_out, si_v, di_v, rs_v, rd_v, pk_v,
                   sem):
    cid = lax.axis_index("c")
    sid = lax.axis_index("s")
    wid = sid * NC + cid
    nch = jnp.where(wid < _PG_CHUNKS % _NW, _PG_CHUNKS // _NW + 1,
                    _PG_CHUNKS // _NW)

    def kbody(k, carry):
        off = (k * _NW + wid) * CH
        pltpu.sync_copy(srcdst.at[pl.ds(off, CH)], si_v)
        pltpu.sync_copy(srcdst.at[pl.ds(E + off, CH)], di_v)
        pltpu.async_copy(pos128.at[si_v], rs_v, sem).wait()
        pltpu.async_copy(pos128.at[di_v], rd_v, sem).wait()

        def pack_row(i, carry2):
            r8 = i // 8
            c16 = (i % 8) * 16
            pk_v[r8, pl.ds(c16, 16)] = (rs_v[i, pl.ds(0, 16)]
                                        - rd_v[i, pl.ds(0, 16)])
            return carry2

        lax.fori_loop(0, CH, pack_row, 0)
        ro = pl.multiple_of((k * _NW + wid) * (CH // 8), 8)
        pltpu.sync_copy(pk_v, vec_out.at[pl.ds(ro, CH // 8)])
        return carry

    lax.fori_loop(0, nch, kbody, 0)


@functools.cache
def _edge_vec():
    return pl.kernel(
        _edge_vec_body,
        out_type=jax.ShapeDtypeStruct((E // 8, LW), jnp.float32),
        mesh=plsc.VectorSubcoreMesh(**_MESH),
        scratch_types=[
            pltpu.VMEM((CH,), jnp.int32),
            pltpu.VMEM((CH,), jnp.int32),
            pltpu.VMEM((CH, LW), jnp.float32),
            pltpu.VMEM((CH, LW), jnp.float32),
            pltpu.VMEM((CH // 8, LW), jnp.float32),
            pltpu.SemaphoreType.DMA,
        ],
    )


E2 = E // NC
LCH = 64
L_CHUNKS = E2 // LCH

RS = 624
_RS_EXTRA_OFF = RS * NS
_RS_EXTRA = N - _RS_EXTRA_OFF


def _rows_copy(src_at, dst_at, sid):
    pltpu.sync_copy(src_at(sid * RS, RS), dst_at(sid * RS, RS))

    @pl.when(sid == NS - 1)
    def _():
        pltpu.sync_copy(src_at(_RS_EXTRA_OFF, _RS_EXTRA),
                        dst_at(_RS_EXTRA_OFF, _RS_EXTRA))


def _make_layer_sc(nblk, used):
    nch_lo = L_CHUNKS // NS
    nch_rem = L_CHUNKS % NS
    npairs = nch_lo // 2

    def body(hm3, w3, src3, dst, zeros, out, gi0, gi1, di0, di1, r0, r1, w0,
             w1, agg_sp, sg0, sg1, sw0, sw1):
        cid = lax.axis_index("c")
        sid = lax.axis_index("s")
        nch = jnp.where(sid < nch_rem, nch_lo + 1, nch_lo)
        gis, dis, rows, ws = (gi0, gi1), (di0, di1), (r0, r1), (w0, w1)
        sgs, sws = (sg0, sg1), (sw0, sw1)

        for cb in range(nblk):
            nlane = (used[cb] + 15) // 16

            def issue(k, b, _cb=cb):
                off = cid * E2 + (k * NS + sid) * LCH
                pltpu.sync_copy(src3.at[pl.ds(_cb * E + off, LCH)], gis[b])
                pltpu.sync_copy(dst.at[pl.ds(off, LCH)], dis[b])
                pltpu.async_copy(hm3.at[gis[b]], rows[b], sgs[b])
                pltpu.async_copy(w3.at[pl.ds(_cb * E + off, LCH)], ws[b],
                                 sws[b])

            def consume(b, _nlane=nlane):
                pltpu.make_async_copy(hm3.at[pl.ds(0, LCH)], rows[b],
                                      sgs[b]).wait()
                pltpu.make_async_copy(w3.at[pl.ds(0, LCH)], ws[b],
                                      sws[b]).wait()

                def mul_row(i, carry2):
                    for j in range(_nlane):
                        sl = pl.ds(j * 16, 16)
                        rows[b][i, sl] = rows[b][i, sl] * ws[b][i, sl]
                    return carry2

                lax.fori_loop(0, LCH, mul_row, 0)
                pltpu.sync_copy(rows[b], agg_sp.at[dis[b]], add=True)

            _rows_copy(lambda o, n: zeros.at[pl.ds(o, n)],
                       lambda o, n: agg_sp.at[pl.ds(o, n)], sid)
            plsc.subcore_barrier()

            issue(0, 0)
            issue(1, 1)

            def kbody(k2, carry):
                for b in range(2):
                    k = k2 * 2 + b
                    consume(b)

                    @pl.when(k + 2 < nch)
                    def _():
                        issue(k + 2, b)
                return carry

            lax.fori_loop(0, npairs, kbody, 0)

            @pl.when(sid < nch_rem)
            def _():
                consume(0)

            plsc.subcore_barrier()
            obase = (cid * nblk + cb) * N
            _rows_copy(lambda o, n: agg_sp.at[pl.ds(o, n)],
                       lambda o, n: out.at[pl.ds(obase + o, n)], sid)
            plsc.subcore_barrier()

    return pl.kernel(
        body,
        out_type=jax.ShapeDtypeStruct((NC * nblk * N, LW), jnp.float32),
        mesh=plsc.VectorSubcoreMesh(**_MESH),
        scratch_types=[
            pltpu.VMEM((LCH,), jnp.int32),
            pltpu.VMEM((LCH,), jnp.int32),
            pltpu.VMEM((LCH,), jnp.int32),
            pltpu.VMEM((LCH,), jnp.int32),
            pltpu.VMEM((LCH, LW), jnp.float32),
            pltpu.VMEM((LCH, LW), jnp.float32),
            pltpu.VMEM((LCH, LW), jnp.float32),
            pltpu.VMEM((LCH, LW), jnp.float32),
            pltpu.VMEM_SHARED((N, LW), jnp.float32),
            pltpu.SemaphoreType.DMA,
            pltpu.SemaphoreType.DMA,
            pltpu.SemaphoreType.DMA,
            pltpu.SemaphoreType.DMA,
        ],
    )


@functools.cache
def _layer_sc(do):
    nblk, used = _PACK[do]
    return _make_layer_sc(nblk, used)


BE = 3200

_SQ3 = float(np.sqrt(3.0))
_SQ15 = float(np.sqrt(15.0))
_SQ5 = float(np.sqrt(5.0))
_EMB_STEP = float(MAXR / (NB + 1))
_EMB_SCALE = float(NB ** 0.5)


def _geom_body(vecp_ref, eattr_ref, ea_ref, emb_ref):
    vec = vecp_ref[:, :3]
    norm = jnp.sqrt(jnp.sum(vec * vec, axis=1, keepdims=True) + EPS)
    v = vec / norm
    x_, y_, z_ = v[:, 0], v[:, 1], v[:, 2]
    sh2 = jnp.stack([
        _SQ15 * x_ * z_,
        _SQ15 * x_ * y_,
        _SQ5 * (y_ * y_ - 0.5 * (x_ * x_ + z_ * z_)),
        _SQ15 * y_ * z_,
        (_SQ15 / 2.0) * (z_ * z_ - x_ * x_),
    ], axis=1)
    ea_ref[...] = jnp.concatenate(
        [eattr_ref[...], jnp.ones_like(norm), _SQ3 * v, sh2], axis=1)
    values = (lax.broadcasted_iota(jnp.int32, (1, NB), 1).astype(jnp.float32)
              + 1.0) * _EMB_STEP
    diff = (norm - values) / _EMB_STEP
    y = jnp.cos(0.5 * np.pi * diff)
    mask = ((diff > -1.0) & (diff < 1.0)).astype(jnp.float32)
    emb_ref[...] = y * mask * _EMB_SCALE


def _geom(vecp, edge_attr):
    return pl.pallas_call(
        _geom_body,
        grid=(E // BE,),
        in_specs=[
            pl.BlockSpec((BE, 16), lambda i: (i, 0)),
            pl.BlockSpec((BE, D_EDGE), lambda i: (i, 0)),
        ],
        out_specs=[
            pl.BlockSpec((BE, D_EDGE + 9), lambda i: (i, 0)),
            pl.BlockSpec((BE, NB), lambda i: (i, 0)),
        ],
        out_shape=[
            jax.ShapeDtypeStruct((E, D_EDGE + 9), jnp.float32),
            jax.ShapeDtypeStruct((E, NB), jnp.float32),
        ],
    )(vecp, edge_attr)


def _ew_body(emb_ref, ea_ref, w1_ref, b1_ref, w2_ref, b2_ref, wr_ref, ws_ref,
             w_out):
    nblk, _, _ = w_out.shape
    do = wr_ref.shape[1]
    r = jax.nn.silu(emb_ref[...] @ w1_ref[...] + b1_ref[...])
    r = jax.nn.silu(r @ w2_ref[...] + b2_ref[...])
    r = r @ wr_ref[...]
    w = (ea_ref[...] @ ws_ref[...]) * r
    if nblk * LW != do:
        w = jnp.pad(w, ((0, 0), (0, nblk * LW - do)))
    for q in range(nblk):
        w_out[q] = w[:, q * LW:(q + 1) * LW]


def _edge_weights(emb, ea_full, p, i):
    do = DIMS[i + 1]
    nblk, _ = _PACK[do]
    full = lambda i_: (0, 0)
    return pl.pallas_call(
        _ew_body,
        grid=(E // BE,),
        in_specs=[
            pl.BlockSpec((BE, NB), lambda i_: (i_, 0)),
            pl.BlockSpec((BE, D_EDGE + 9), lambda i_: (i_, 0)),
            pl.BlockSpec((NB, MUL), full),
            pl.BlockSpec((1, MUL), full),
            pl.BlockSpec((MUL, 2 * MUL), full),
            pl.BlockSpec((1, 2 * MUL), full),
            pl.BlockSpec((2 * MUL, do), full),
            pl.BlockSpec((D_EDGE + 9, do), full),
        ],
        out_specs=pl.BlockSpec((nblk, BE, LW), lambda i_: (0, i_, 0)),
        out_shape=jax.ShapeDtypeStruct((nblk, E, LW), jnp.float32),
    )(emb, ea_full, p['W1_%d' % i], p['b1_%d' % i].reshape(1, -1),
      p['W2_%d' % i], p['b2_%d' % i].reshape(1, -1), p['W_rad_%d' % i],
      p['W_sh_%d' % i])


BN = 2000
_INV_SQRT_NEIGH = float(1.0 / np.sqrt(NUM_NEIGH))


def _sum_agg(agg_ref, do):
    nblk, _ = _PACK[do]
    cols = []
    for q in range(nblk):
        cols.append(agg_ref[0, q] + agg_ref[1, q])
    agg = jnp.concatenate(cols, axis=1) if nblk > 1 else cols[0]
    return agg[:, :do]


def _pack_hm(hm, nblk, hm_out):
    do = hm.shape[1]
    if nblk * LW != do:
        hm = jnp.pad(hm, ((0, 0), (0, nblk * LW - do)))
    for q in range(nblk):
        hm_out[q] = hm[:, q * LW:(q + 1) * LW]


def _node0_body(x_ref, wm_ref, hm_out):
    _pack_hm(x_ref[...] @ wm_ref[...], hm_out.shape[0], hm_out)


def _node0(x, wm):
    do = wm.shape[1]
    nblk, _ = _PACK[do]
    return pl.pallas_call(
        _node0_body,
        grid=(N // BN,),
        in_specs=[
            pl.BlockSpec((BN, D_IN), lambda i: (i, 0)),
            pl.BlockSpec((D_IN, do), lambda i: (0, 0)),
        ],
        out_specs=pl.BlockSpec((nblk, BN, LW), lambda i: (0, i, 0)),
        out_shape=jax.ShapeDtypeStruct((nblk, N, LW), jnp.float32),
    )(x, wm)


def _node_post_body(agg_ref, h_ref, na_ref, wself_ref, wattr_ref, wm_ref,
                    h_out, hm_out):
    do = wself_ref.shape[1]
    agg = _sum_agg(agg_ref, do)
    hn = (agg * _INV_SQRT_NEIGH + h_ref[...] @ wself_ref[...]
          + na_ref[...] @ wattr_ref[...])
    hn = jax.nn.silu(hn)
    h_out[...] = hn
    _pack_hm(hn @ wm_ref[...], hm_out.shape[0], hm_out)


def _node_post(agg4, h, na, wself, wattr, wm_next):
    di, do = wself.shape
    nblk, _ = _PACK[do]
    don = wm_next.shape[1]
    nblkn, _ = _PACK[don]
    full = lambda i: (0, 0)
    return pl.pallas_call(
        _node_post_body,
        grid=(N // BN,),
        in_specs=[
            pl.BlockSpec((NC, nblk, BN, LW), lambda i: (0, 0, i, 0)),
            pl.BlockSpec((BN, di), lambda i: (i, 0)),
            pl.BlockSpec((BN, D_ATTR), lambda i: (i, 0)),
            pl.BlockSpec((di, do), full),
            pl.BlockSpec((D_ATTR, do), full),
            pl.BlockSpec((do, don), full),
        ],
        out_specs=[
            pl.BlockSpec((BN, do), lambda i: (i, 0)),
            pl.BlockSpec((nblkn, BN, LW), lambda i: (0, i, 0)),
        ],
        out_shape=[
            jax.ShapeDtypeStruct((N, do), jnp.float32),
            jax.ShapeDtypeStruct((nblkn, N, LW), jnp.float32),
        ],
    )(agg4, h, na, wself, wattr, wm_next)


def _node_final_body(agg_ref, h_ref, na_ref, wself_ref, wattr_ref, h_out):
    do = wself_ref.shape[1]
    agg = _sum_agg(agg_ref, do)
    h_out[...] = (agg * _INV_SQRT_NEIGH + h_ref[...] @ wself_ref[...]
                  + na_ref[...] @ wattr_ref[...])


def _node_final(agg4, h, na, wself, wattr):
    di, do = wself.shape
    nblk, _ = _PACK[do]
    full = lambda i: (0, 0)
    return pl.pallas_call(
        _node_final_body,
        grid=(N // BN,),
        in_specs=[
            pl.BlockSpec((NC, nblk, BN, LW), lambda i: (0, 0, i, 0)),
            pl.BlockSpec((BN, di), lambda i: (i, 0)),
            pl.BlockSpec((BN, D_ATTR), lambda i: (i, 0)),
            pl.BlockSpec((di, do), full),
            pl.BlockSpec((D_ATTR, do), full),
        ],
        out_specs=pl.BlockSpec((BN, do), lambda i: (i, 0)),
        out_shape=jax.ShapeDtypeStruct((N, do), jnp.float32),
    )(agg4, h, na, wself, wattr)


def kernel(x, pos, node_attr, edge_attr, params, edge_index, batch):
    src = edge_index[0]
    dst = edge_index[1]
    pos128 = jnp.pad(pos, ((0, 0), (0, LW - 3)))
    src3 = jnp.concatenate([src, src + N, src + 2 * N])
    zeros = jnp.zeros((N, LW), jnp.float32)

    vecp = _edge_vec()(pos128, edge_index.reshape(2 * E))
    ea_full, emb = _geom(vecp.reshape(E, 16), edge_attr)

    h = x
    hm = _node0(x, params['W_msg_0'])
    for i in range(4):
        do = DIMS[i + 1]
        nblk, _ = _PACK[do]
        w3 = _edge_weights(emb, ea_full, params, i)
        agg = _layer_sc(do)(hm.reshape(nblk * N, LW),
                            w3.reshape(nblk * E, LW), src3, dst, zeros)
        agg4 = agg.reshape(NC, nblk, N, LW)
        if i < 3:
            h, hm = _node_post(agg4, h, node_attr, params['W_self_%d' % i],
                               params['W_attr_%d' % i],
                               params['W_msg_%d' % (i + 1)])
        else:
            h = _node_final(agg4, h, node_attr, params['W_self_%d' % i],
                            params['W_attr_%d' % i])

    return (h, node_attr, src, dst, ea_full, emb, batch)


# speedup vs baseline: 2.5752x; 1.0176x over previous
import functools

import jax
import jax.numpy as jnp
import numpy as np
from jax import lax
from jax.experimental import pallas as pl
from jax.experimental.pallas import tpu as pltpu
from jax.experimental.pallas import tpu_sc as plsc

N = 10000
E = 320000
D_IN = 128
D_ATTR = 16
D_EDGE = 4
D_OUT = 64
MUL = 64
HID = 288
NB = 10
MAXR = 5.0
NUM_NEIGH = 32.0
DIMS = [D_IN, HID, HID, HID, D_OUT]
EPS = 1e-12

NC = 2
NS = 16
CH = 128
LW = 128

_PACK = {HID: (3, (128, 128, 32)), D_OUT: (1, (64,))}

_MESH = dict(core_axis_name="c", subcore_axis_name="s", num_cores=NC,
             num_subcores=NS)


_NW = NC * NS
_PG_CHUNKS = E // CH


def _edge_vec_body(pos128, srcdst, vec_out, si_v, di_v, rs_v, rd_v, pk_v,
                   sem, sem2):
    cid = lax.axis_index("c")
    sid = lax.axis_index("s")
    wid = sid * NC + cid
    nch = jnp.where(wid < _PG_CHUNKS % _NW, _PG_CHUNKS // _NW + 1,
                    _PG_CHUNKS // _NW)

    def kbody(k, carry):
        off = (k * _NW + wid) * CH
        pltpu.sync_copy(srcdst.at[pl.ds(off, CH)], si_v)
        pltpu.sync_copy(srcdst.at[pl.ds(E + off, CH)], di_v)
        pltpu.async_copy(pos128.at[si_v], rs_v, sem)
        pltpu.async_copy(pos128.at[di_v], rd_v, sem2)
        pltpu.make_async_copy(pos128.at[pl.ds(0, CH)], rs_v, sem).wait()
        pltpu.make_async_copy(pos128.at[pl.ds(0, CH)], rd_v, sem2).wait()

        def pack_row(r8, carry2):
            for m in range(8):
                i = r8 * 8 + m
                pk_v[r8, pl.ds(m * 16, 16)] = (rs_v[i, pl.ds(0, 16)]
                                               - rd_v[i, pl.ds(0, 16)])
            return carry2

        lax.fori_loop(0, CH // 8, pack_row, 0)
        ro = pl.multiple_of((k * _NW + wid) * (CH // 8), 8)
        pltpu.sync_copy(pk_v, vec_out.at[pl.ds(ro, CH // 8)])
        return carry

    lax.fori_loop(0, nch, kbody, 0)


@functools.cache
def _edge_vec():
    return pl.kernel(
        _edge_vec_body,
        out_type=jax.ShapeDtypeStruct((E // 8, LW), jnp.float32),
        mesh=plsc.VectorSubcoreMesh(**_MESH),
        scratch_types=[
            pltpu.VMEM((CH,), jnp.int32),
            pltpu.VMEM((CH,), jnp.int32),
            pltpu.VMEM((CH, LW), jnp.float32),
            pltpu.VMEM((CH, LW), jnp.float32),
            pltpu.VMEM((CH // 8, LW), jnp.float32),
            pltpu.SemaphoreType.DMA,
            pltpu.SemaphoreType.DMA,
        ],
    )


E2 = E // NC
LCH = 64
L_CHUNKS = E2 // LCH

RS = 624
_RS_EXTRA_OFF = RS * NS
_RS_EXTRA = N - _RS_EXTRA_OFF


def _rows_copy(src_at, dst_at, sid):
    pltpu.sync_copy(src_at(sid * RS, RS), dst_at(sid * RS, RS))

    @pl.when(sid == NS - 1)
    def _():
        pltpu.sync_copy(src_at(_RS_EXTRA_OFF, _RS_EXTRA),
                        dst_at(_RS_EXTRA_OFF, _RS_EXTRA))


def _make_layer_sc(nblk, used):
    nch_lo = L_CHUNKS // NS
    nch_rem = L_CHUNKS % NS
    npairs = nch_lo // 2

    def body(hm3, w3, src3, dst, zeros, out, gi0, gi1, di0, di1, r0, r1, w0,
             w1, agg_sp, sg0, sg1, sw0, sw1):
        cid = lax.axis_index("c")
        sid = lax.axis_index("s")
        nch = jnp.where(sid < nch_rem, nch_lo + 1, nch_lo)
        gis, dis, rows, ws = (gi0, gi1), (di0, di1), (r0, r1), (w0, w1)
        sgs, sws = (sg0, sg1), (sw0, sw1)

        for cb in range(nblk):
            nlane = (used[cb] + 15) // 16

            def issue(k, b, _cb=cb):
                off = cid * E2 + (k * NS + sid) * LCH
                pltpu.sync_copy(src3.at[pl.ds(_cb * E + off, LCH)], gis[b])
                pltpu.sync_copy(dst.at[pl.ds(off, LCH)], dis[b])
                pltpu.async_copy(hm3.at[gis[b]], rows[b], sgs[b])
                pltpu.async_copy(w3.at[pl.ds(_cb * E + off, LCH)], ws[b],
                                 sws[b])

            def consume(b, _nlane=nlane):
                pltpu.make_async_copy(hm3.at[pl.ds(0, LCH)], rows[b],
                                      sgs[b]).wait()
                pltpu.make_async_copy(w3.at[pl.ds(0, LCH)], ws[b],
                                      sws[b]).wait()

                def mul_row(i4, carry2):
                    for ii in range(4):
                        i = i4 * 4 + ii
                        for j in range(_nlane):
                            sl = pl.ds(j * 16, 16)
                            rows[b][i, sl] = rows[b][i, sl] * ws[b][i, sl]
                    return carry2

                lax.fori_loop(0, LCH // 4, mul_row, 0)
                pltpu.sync_copy(rows[b], agg_sp.at[dis[b]], add=True)

            _rows_copy(lambda o, n: zeros.at[pl.ds(o, n)],
                       lambda o, n: agg_sp.at[pl.ds(o, n)], sid)
            plsc.subcore_barrier()

            issue(0, 0)
            issue(1, 1)

            def kbody(k2, carry):
                for b in range(2):
                    k = k2 * 2 + b
                    consume(b)

                    @pl.when(k + 2 < nch)
                    def _():
                        issue(k + 2, b)
                return carry

            lax.fori_loop(0, npairs, kbody, 0)

            @pl.when(sid < nch_rem)
            def _():
                consume(0)

            plsc.subcore_barrier()
            obase = (cid * nblk + cb) * N
            _rows_copy(lambda o, n: agg_sp.at[pl.ds(o, n)],
                       lambda o, n: out.at[pl.ds(obase + o, n)], sid)
            plsc.subcore_barrier()

    return pl.kernel(
        body,
        out_type=jax.ShapeDtypeStruct((NC * nblk * N, LW), jnp.float32),
        mesh=plsc.VectorSubcoreMesh(**_MESH),
        scratch_types=[
            pltpu.VMEM((LCH,), jnp.int32),
            pltpu.VMEM((LCH,), jnp.int32),
            pltpu.VMEM((LCH,), jnp.int32),
            pltpu.VMEM((LCH,), jnp.int32),
            pltpu.VMEM((LCH, LW), jnp.float32),
            pltpu.VMEM((LCH, LW), jnp.float32),
            pltpu.VMEM((LCH, LW), jnp.float32),
            pltpu.VMEM((LCH, LW), jnp.float32),
            pltpu.VMEM_SHARED((N, LW), jnp.float32),
            pltpu.SemaphoreType.DMA,
            pltpu.SemaphoreType.DMA,
            pltpu.SemaphoreType.DMA,
            pltpu.SemaphoreType.DMA,
        ],
    )


@functools.cache
def _layer_sc(do):
    nblk, used = _PACK[do]
    return _make_layer_sc(nblk, used)


BE = 3200

_SQ3 = float(np.sqrt(3.0))
_SQ15 = float(np.sqrt(15.0))
_SQ5 = float(np.sqrt(5.0))
_EMB_STEP = float(MAXR / (NB + 1))
_EMB_SCALE = float(NB ** 0.5)


def _geom_body(vecp_ref, eattr_ref, ea_ref, emb_ref):
    vec = vecp_ref[:, :3]
    norm = jnp.sqrt(jnp.sum(vec * vec, axis=1, keepdims=True) + EPS)
    v = vec / norm
    x_, y_, z_ = v[:, 0], v[:, 1], v[:, 2]
    sh2 = jnp.stack([
        _SQ15 * x_ * z_,
        _SQ15 * x_ * y_,
        _SQ5 * (y_ * y_ - 0.5 * (x_ * x_ + z_ * z_)),
        _SQ15 * y_ * z_,
        (_SQ15 / 2.0) * (z_ * z_ - x_ * x_),
    ], axis=1)
    ea_ref[...] = jnp.concatenate(
        [eattr_ref[...], jnp.ones_like(norm), _SQ3 * v, sh2], axis=1)
    values = (lax.broadcasted_iota(jnp.int32, (1, NB), 1).astype(jnp.float32)
              + 1.0) * _EMB_STEP
    diff = (norm - values) / _EMB_STEP
    y = jnp.cos(0.5 * np.pi * diff)
    mask = ((diff > -1.0) & (diff < 1.0)).astype(jnp.float32)
    emb_ref[...] = y * mask * _EMB_SCALE


def _geom(vecp, edge_attr):
    return pl.pallas_call(
        _geom_body,
        grid=(E // BE,),
        in_specs=[
            pl.BlockSpec((BE, 16), lambda i: (i, 0)),
            pl.BlockSpec((BE, D_EDGE), lambda i: (i, 0)),
        ],
        out_specs=[
            pl.BlockSpec((BE, D_EDGE + 9), lambda i: (i, 0)),
            pl.BlockSpec((BE, NB), lambda i: (i, 0)),
        ],
        out_shape=[
            jax.ShapeDtypeStruct((E, D_EDGE + 9), jnp.float32),
            jax.ShapeDtypeStruct((E, NB), jnp.float32),
        ],
    )(vecp, edge_attr)


def _ew_body(emb_ref, ea_ref, w1_ref, b1_ref, w2_ref, b2_ref, wr_ref, ws_ref,
             w_out):
    nblk, _, _ = w_out.shape
    do = wr_ref.shape[1]
    r = jax.nn.silu(emb_ref[...] @ w1_ref[...] + b1_ref[...])
    r = jax.nn.silu(r @ w2_ref[...] + b2_ref[...])
    r = r @ wr_ref[...]
    w = (ea_ref[...] @ ws_ref[...]) * r
    if nblk * LW != do:
        w = jnp.pad(w, ((0, 0), (0, nblk * LW - do)))
    for q in range(nblk):
        w_out[q] = w[:, q * LW:(q + 1) * LW]


def _edge_weights(emb, ea_full, p, i):
    do = DIMS[i + 1]
    nblk, _ = _PACK[do]
    full = lambda i_: (0, 0)
    return pl.pallas_call(
        _ew_body,
        grid=(E // BE,),
        in_specs=[
            pl.BlockSpec((BE, NB), lambda i_: (i_, 0)),
            pl.BlockSpec((BE, D_EDGE + 9), lambda i_: (i_, 0)),
            pl.BlockSpec((NB, MUL), full),
            pl.BlockSpec((1, MUL), full),
            pl.BlockSpec((MUL, 2 * MUL), full),
            pl.BlockSpec((1, 2 * MUL), full),
            pl.BlockSpec((2 * MUL, do), full),
            pl.BlockSpec((D_EDGE + 9, do), full),
        ],
        out_specs=pl.BlockSpec((nblk, BE, LW), lambda i_: (0, i_, 0)),
        out_shape=jax.ShapeDtypeStruct((nblk, E, LW), jnp.float32),
    )(emb, ea_full, p['W1_%d' % i], p['b1_%d' % i].reshape(1, -1),
      p['W2_%d' % i], p['b2_%d' % i].reshape(1, -1), p['W_rad_%d' % i],
      p['W_sh_%d' % i])


BN = 2000
_INV_SQRT_NEIGH = float(1.0 / np.sqrt(NUM_NEIGH))


def _sum_agg(agg_ref, do):
    nblk, _ = _PACK[do]
    cols = []
    for q in range(nblk):
        cols.append(agg_ref[0, q] + agg_ref[1, q])
    agg = jnp.concatenate(cols, axis=1) if nblk > 1 else cols[0]
    return agg[:, :do]


def _pack_hm(hm, nblk, hm_out):
    do = hm.shape[1]
    if nblk * LW != do:
        hm = jnp.pad(hm, ((0, 0), (0, nblk * LW - do)))
    for q in range(nblk):
        hm_out[q] = hm[:, q * LW:(q + 1) * LW]


def _node0_body(x_ref, wm_ref, hm_out):
    _pack_hm(x_ref[...] @ wm_ref[...], hm_out.shape[0], hm_out)


def _node0(x, wm):
    do = wm.shape[1]
    nblk, _ = _PACK[do]
    return pl.pallas_call(
        _node0_body,
        grid=(N // BN,),
        in_specs=[
            pl.BlockSpec((BN, D_IN), lambda i: (i, 0)),
            pl.BlockSpec((D_IN, do), lambda i: (0, 0)),
        ],
        out_specs=pl.BlockSpec((nblk, BN, LW), lambda i: (0, i, 0)),
        out_shape=jax.ShapeDtypeStruct((nblk, N, LW), jnp.float32),
    )(x, wm)


def _node_post_body(agg_ref, h_ref, na_ref, wself_ref, wattr_ref, wm_ref,
                    h_out, hm_out):
    do = wself_ref.shape[1]
    agg = _sum_agg(agg_ref, do)
    hn = (agg * _INV_SQRT_NEIGH + h_ref[...] @ wself_ref[...]
          + na_ref[...] @ wattr_ref[...])
    hn = jax.nn.silu(hn)
    h_out[...] = hn
    _pack_hm(hn @ wm_ref[...], hm_out.shape[0], hm_out)


def _node_post(agg4, h, na, wself, wattr, wm_next):
    di, do = wself.shape
    nblk, _ = _PACK[do]
    don = wm_next.shape[1]
    nblkn, _ = _PACK[don]
    full = lambda i: (0, 0)
    return pl.pallas_call(
        _node_post_body,
        grid=(N // BN,),
        in_specs=[
            pl.BlockSpec((NC, nblk, BN, LW), lambda i: (0, 0, i, 0)),
            pl.BlockSpec((BN, di), lambda i: (i, 0)),
            pl.BlockSpec((BN, D_ATTR), lambda i: (i, 0)),
            pl.BlockSpec((di, do), full),
            pl.BlockSpec((D_ATTR, do), full),
            pl.BlockSpec((do, don), full),
        ],
        out_specs=[
            pl.BlockSpec((BN, do), lambda i: (i, 0)),
            pl.BlockSpec((nblkn, BN, LW), lambda i: (0, i, 0)),
        ],
        out_shape=[
            jax.ShapeDtypeStruct((N, do), jnp.float32),
            jax.ShapeDtypeStruct((nblkn, N, LW), jnp.float32),
        ],
    )(agg4, h, na, wself, wattr, wm_next)


def _node_final_body(agg_ref, h_ref, na_ref, wself_ref, wattr_ref, h_out):
    do = wself_ref.shape[1]
    agg = _sum_agg(agg_ref, do)
    h_out[...] = (agg * _INV_SQRT_NEIGH + h_ref[...] @ wself_ref[...]
                  + na_ref[...] @ wattr_ref[...])


def _node_final(agg4, h, na, wself, wattr):
    di, do = wself.shape
    nblk, _ = _PACK[do]
    full = lambda i: (0, 0)
    return pl.pallas_call(
        _node_final_body,
        grid=(N // BN,),
        in_specs=[
            pl.BlockSpec((NC, nblk, BN, LW), lambda i: (0, 0, i, 0)),
            pl.BlockSpec((BN, di), lambda i: (i, 0)),
            pl.BlockSpec((BN, D_ATTR), lambda i: (i, 0)),
            pl.BlockSpec((di, do), full),
            pl.BlockSpec((D_ATTR, do), full),
        ],
        out_specs=pl.BlockSpec((BN, do), lambda i: (i, 0)),
        out_shape=jax.ShapeDtypeStruct((N, do), jnp.float32),
    )(agg4, h, na, wself, wattr)


def kernel(x, pos, node_attr, edge_attr, params, edge_index, batch):
    src = edge_index[0]
    dst = edge_index[1]
    pos128 = jnp.pad(pos, ((0, 0), (0, LW - 3)))
    src3 = jnp.concatenate([src, src + N, src + 2 * N])
    zeros = jnp.zeros((N, LW), jnp.float32)

    vecp = _edge_vec()(pos128, edge_index.reshape(2 * E))
    ea_full, emb = _geom(vecp.reshape(E, 16), edge_attr)

    h = x
    hm = _node0(x, params['W_msg_0'])
    for i in range(4):
        do = DIMS[i + 1]
        nblk, _ = _PACK[do]
        w3 = _edge_weights(emb, ea_full, params, i)
        agg = _layer_sc(do)(hm.reshape(nblk * N, LW),
                            w3.reshape(nblk * E, LW), src3, dst, zeros)
        agg4 = agg.reshape(NC, nblk, N, LW)
        if i < 3:
            h, hm = _node_post(agg4, h, node_attr, params['W_self_%d' % i],
                               params['W_attr_%d' % i],
                               params['W_msg_%d' % (i + 1)])
        else:
            h = _node_final(agg4, h, node_attr, params['W_self_%d' % i],
                            params['W_attr_%d' % i])

    return (h, node_attr, src, dst, ea_full, emb, batch)
